# baseline (speedup 1.0000x reference)
import jax
import jax.numpy as jnp
from jax.experimental import pallas as pl

_WAY = 5


def _mul_dup_body(x_ref, s_ref, o_ref):
    y = x_ref[...] * s_ref[...]
    o_ref[:, 0] = y[:, None]
    o_ref[:, 1] = y[:, None]


def kernel(x, s_ca, shuffle_num):
    N, c, h, w = x.shape
    hw = h * w
    G = N // _WAY
    x3 = x.reshape(N, c, hw)
    s3 = s_ca.reshape(N, c, 1)

    out = pl.pallas_call(
        _mul_dup_body,
        grid=(N,),
        in_specs=[
            pl.BlockSpec((1, c, hw), lambda i: (i, 0, 0)),
            pl.BlockSpec((1, c, 1), lambda i: (i, 0, 0)),
        ],
        out_specs=pl.BlockSpec((1, 2, 1, c, hw),
                               lambda i: (i // G, 0, i % G, 0, 0)),
        out_shape=jax.ShapeDtypeStruct((_WAY, 2, G, c, hw), x.dtype),
    )(x3, s3)
    return out.reshape(2 * N, c, h, w)
